# baseline (speedup 1.0000x reference)
import functools

import jax
import jax.numpy as jnp
from jax import lax
from jax.experimental import pallas as pl
from jax.experimental.pallas import tpu as pltpu
from jax.experimental.pallas import tpu_sc as plsc

_NC, _NS = 2, 16
_NW = _NC * _NS

_CHUNK = 128
_K = 4
_GROUP = _CHUNK * _K


def _make(num_idx_rows, d):
    b = num_idx_rows * _CHUNK
    b_per_w = b // _NW
    groups = b_per_w // _GROUP
    rows_per_w = num_idx_rows // _NW

    mesh = plsc.VectorSubcoreMesh(
        core_axis_name="c", subcore_axis_name="s",
        num_cores=_NC, num_subcores=_NS)

    @functools.partial(
        pl.kernel,
        mesh=mesh,
        compiler_params=pltpu.CompilerParams(use_tc_tiling_on_sc=False),
        out_type=jax.ShapeDtypeStruct((b, d), jnp.float32),
        scratch_types=[
            pltpu.VMEM((_K, _CHUNK), jnp.int32),
            pltpu.VMEM((_GROUP, d), jnp.float32),
            pltpu.SemaphoreType.DMA,
        ],
    )
    def gather_kernel(idx_hbm, table_hbm, out_hbm, idx_v, rows_v, sem):
        wid = lax.axis_index("s") * _NC + lax.axis_index("c")
        row_base = wid * rows_per_w
        out_base = wid * b_per_w

        def body(g, carry):
            pltpu.sync_copy(idx_hbm.at[pl.ds(row_base + g * _K, _K)], idx_v)
            copies = [
                pltpu.async_copy(
                    table_hbm.at[idx_v.at[j]],
                    rows_v.at[pl.ds(j * _CHUNK, _CHUNK)],
                    sem,
                )
                for j in range(_K)
            ]
            for c in copies:
                c.wait()
            pltpu.sync_copy(
                rows_v, out_hbm.at[pl.ds(out_base + g * _GROUP, _GROUP)])
            return carry

        lax.fori_loop(0, groups, body, 0)

    return gather_kernel


def kernel(input_ids, word_embeddings):
    batch, seq = input_ids.shape
    _, d = word_embeddings.shape
    b = batch * seq
    idx2d = input_ids.reshape(b // _CHUNK, _CHUNK).astype(jnp.int32)
    out = _make(b // _CHUNK, d)(idx2d, word_embeddings)
    return out.reshape(batch, seq, d)


# speedup vs baseline: 1.0426x; 1.0426x over previous
import functools

import jax
import jax.numpy as jnp
from jax import lax
from jax.experimental import pallas as pl
from jax.experimental.pallas import tpu as pltpu
from jax.experimental.pallas import tpu_sc as plsc

_NC, _NS = 2, 16
_NW = _NC * _NS

_CHUNK = 128
_K = 4
_GROUP = _CHUNK * _K


def _make(num_idx_rows, d):
    b = num_idx_rows * _CHUNK
    b_per_w = b // _NW
    groups = b_per_w // _GROUP
    rows_per_w = num_idx_rows // _NW

    mesh = plsc.VectorSubcoreMesh(
        core_axis_name="c", subcore_axis_name="s",
        num_cores=_NC, num_subcores=_NS)

    @functools.partial(
        pl.kernel,
        mesh=mesh,
        compiler_params=pltpu.CompilerParams(use_tc_tiling_on_sc=False),
        out_type=jax.ShapeDtypeStruct((b, d), jnp.float32),
        scratch_types=[
            pltpu.VMEM((rows_per_w, _CHUNK), jnp.int32),
            pltpu.VMEM((2, _GROUP, d), jnp.float32),
            pltpu.SemaphoreType.DMA,
            pltpu.SemaphoreType.DMA,
            pltpu.SemaphoreType.DMA,
            pltpu.SemaphoreType.DMA,
        ],
    )
    def gather_kernel(idx_hbm, table_hbm, out_hbm, idx_all, rows_v,
                      gsem0, gsem1, osem0, osem1):
        wid = lax.axis_index("s") * _NC + lax.axis_index("c")
        row_base = wid * rows_per_w
        out_base = wid * b_per_w

        pltpu.sync_copy(idx_hbm.at[pl.ds(row_base, rows_per_w)], idx_all)

        gsems = (gsem0, gsem1)
        osems = (osem0, osem1)

        def fire(g, s):
            for j in range(_K):
                pltpu.async_copy(
                    table_hbm.at[idx_all.at[g * _K + j]],
                    rows_v.at[s].at[pl.ds(j * _CHUNK, _CHUNK)],
                    gsems[s])

        def wait_gathers(s):
            pltpu.make_async_copy(
                table_hbm.at[pl.ds(0, _GROUP)], rows_v.at[s],
                gsems[s]).wait()

        def fire_out(g, s):
            pltpu.async_copy(
                rows_v.at[s],
                out_hbm.at[pl.ds(out_base + g * _GROUP, _GROUP)],
                osems[s])

        def wait_out(g, s):
            pltpu.make_async_copy(
                rows_v.at[s],
                out_hbm.at[pl.ds(out_base + g * _GROUP, _GROUP)],
                osems[s]).wait()

        fire(0, 0)
        wait_gathers(0)
        fire_out(0, 0)
        fire(1, 1)

        def body(i, carry):
            g1 = 2 * i + 1
            wait_gathers(1)
            fire_out(g1, 1)
            wait_out(g1 - 1, 0)
            fire(g1 + 1, 0)
            g2 = 2 * i + 2
            wait_gathers(0)
            fire_out(g2, 0)
            wait_out(g2 - 1, 1)
            fire(g2 + 1, 1)
            return carry

        lax.fori_loop(0, (groups - 2) // 2, body, 0)

        wait_gathers(1)
        fire_out(groups - 1, 1)
        wait_out(groups - 2, 0)
        wait_out(groups - 1, 1)

    return gather_kernel


def kernel(input_ids, word_embeddings):
    batch, seq = input_ids.shape
    _, d = word_embeddings.shape
    b = batch * seq
    idx2d = input_ids.reshape(b // _CHUNK, _CHUNK).astype(jnp.int32)
    out = _make(b // _CHUNK, d)(idx2d, word_embeddings)
    return out.reshape(batch, seq, d)


# speedup vs baseline: 1.0443x; 1.0017x over previous
import functools

import jax
import jax.numpy as jnp
from jax import lax
from jax.experimental import pallas as pl
from jax.experimental.pallas import tpu as pltpu
from jax.experimental.pallas import tpu_sc as plsc

_NC, _NS = 2, 16
_NW = _NC * _NS

_K = 2
_PIECES = ((0, 104), (104, 96))


def _make(batch, seq, d):
    rows_per_w = batch // _NW
    groups = rows_per_w // _K

    mesh = plsc.VectorSubcoreMesh(
        core_axis_name="c", subcore_axis_name="s",
        num_cores=_NC, num_subcores=_NS)

    @functools.partial(
        pl.kernel,
        mesh=mesh,
        compiler_params=pltpu.CompilerParams(use_tc_tiling_on_sc=False),
        out_type=jax.ShapeDtypeStruct((batch, seq, d), jnp.float32),
        scratch_types=[
            pltpu.VMEM((rows_per_w, seq), jnp.int32),
            pltpu.VMEM((2, _K, seq, d), jnp.float32),
            pltpu.SemaphoreType.DMA,
            pltpu.SemaphoreType.DMA,
            pltpu.SemaphoreType.DMA,
            pltpu.SemaphoreType.DMA,
        ],
    )
    def gather_kernel(idx_hbm, table_hbm, out_hbm, idx_all, rows_v,
                      gsem0, gsem1, osem0, osem1):
        wid = lax.axis_index("s") * _NC + lax.axis_index("c")
        base = wid * rows_per_w

        pltpu.sync_copy(idx_hbm.at[pl.ds(base, rows_per_w)], idx_all)

        gsems = (gsem0, gsem1)
        osems = (osem0, osem1)

        def fire(g, s):
            for k in range(_K):
                for off, n in _PIECES:
                    pltpu.async_copy(
                        table_hbm.at[idx_all.at[g * _K + k, pl.ds(off, n)]],
                        rows_v.at[s, k].at[pl.ds(off, n)],
                        gsems[s])

        def wait_gathers(s):
            pltpu.make_async_copy(
                out_hbm.at[pl.ds(0, _K)], rows_v.at[s], gsems[s]).wait()

        def fire_out(g, s):
            pltpu.async_copy(
                rows_v.at[s], out_hbm.at[pl.ds(base + g * _K, _K)], osems[s])

        def wait_out(g, s):
            pltpu.make_async_copy(
                rows_v.at[s], out_hbm.at[pl.ds(base + g * _K, _K)],
                osems[s]).wait()

        fire(0, 0)
        wait_gathers(0)
        fire_out(0, 0)
        fire(1, 1)

        def body(i, carry):
            g1 = 2 * i + 1
            wait_gathers(1)
            fire_out(g1, 1)
            wait_out(g1 - 1, 0)
            fire(g1 + 1, 0)
            g2 = 2 * i + 2
            wait_gathers(0)
            fire_out(g2, 0)
            wait_out(g2 - 1, 1)
            fire(g2 + 1, 1)
            return carry

        lax.fori_loop(0, (groups - 2) // 2, body, 0)

        wait_gathers(1)
        fire_out(groups - 1, 1)
        wait_out(groups - 2, 0)
        wait_out(groups - 1, 1)

    return gather_kernel


def kernel(input_ids, word_embeddings):
    batch, seq = input_ids.shape
    _, d = word_embeddings.shape
    return _make(batch, seq, d)(input_ids.astype(jnp.int32), word_embeddings)
